# baseline (speedup 1.0000x reference)
import functools

import jax
import jax.numpy as jnp
from jax import lax
from jax.experimental import pallas as pl
from jax.experimental.pallas import tpu as pltpu
from jax.experimental.pallas import tpu_sc as plsc

NUM_GRAPH = 4
HID = 32
SIGMA = 1.0
TW = 2 * NUM_GRAPH + 1

NC = 2
NS = 16
NW = NC * NS


def _tc_alr_kernel(f_ref, att_ref, w_ref, b_ref, wl_ref, bl_ref,
                   wr_ref, br_ref, alr_ref):
    f = f_ref[...]
    alr_ref[:, 2 * NUM_GRAPH:] = jnp.zeros_like(alr_ref[:, 2 * NUM_GRAPH:])
    for g in range(NUM_GRAPH):
        fa = f * att_ref[g, :][None, :]
        h = jnp.dot(fa, w_ref[g], preferred_element_type=jnp.float32)
        h = h + b_ref[g, :][None, :]
        alr_ref[:, g:g + 1] = (
            jnp.dot(h, wl_ref[g], preferred_element_type=jnp.float32)
            + bl_ref[g][None, :])
        alr_ref[:, NUM_GRAPH + g:NUM_GRAPH + g + 1] = (
            jnp.dot(h, wr_ref[g], preferred_element_type=jnp.float32)
            + br_ref[g][None, :])


def _tc_hidden_kernel(f_ref, att_ref, w_ref, b_ref, hid_ref):
    f = f_ref[...]
    for g in range(NUM_GRAPH):
        fa = f * att_ref[g, :][None, :]
        h = jnp.dot(fa, w_ref[g], preferred_element_type=jnp.float32)
        hid_ref[:, g * HID:(g + 1) * HID] = h + b_ref[g, :][None, :]


def _alr_part(features, att, W, b, Wl, bl, Wr, br):
    n, d = features.shape
    blk = 1000
    grid = n // blk
    full = lambda *dims: pl.BlockSpec(dims, lambda i: (0,) * len(dims))
    return pl.pallas_call(
        _tc_alr_kernel,
        grid=(grid,),
        in_specs=[
            pl.BlockSpec((blk, d), lambda i: (i, 0)),
            full(NUM_GRAPH, d),
            full(NUM_GRAPH, d, HID),
            full(NUM_GRAPH, HID),
            full(NUM_GRAPH, HID, 1),
            full(NUM_GRAPH, 1),
            full(NUM_GRAPH, HID, 1),
            full(NUM_GRAPH, 1),
        ],
        out_specs=pl.BlockSpec((blk, TW), lambda i: (i, 0)),
        out_shape=jax.ShapeDtypeStruct((n, TW), jnp.float32),
    )(features, att, W, b, Wl, bl, Wr, br)


def _hidden_part(features, att, W, b):
    n, d = features.shape
    blk = 1000
    grid = n // blk
    full = lambda *dims: pl.BlockSpec(dims, lambda i: (0,) * len(dims))
    return pl.pallas_call(
        _tc_hidden_kernel,
        grid=(grid,),
        in_specs=[
            pl.BlockSpec((blk, d), lambda i: (i, 0)),
            full(NUM_GRAPH, d),
            full(NUM_GRAPH, d, HID),
            full(NUM_GRAPH, HID),
        ],
        out_specs=pl.BlockSpec((blk, NUM_GRAPH * HID), lambda i: (i, 0)),
        out_shape=jax.ShapeDtypeStruct((n, NUM_GRAPH * HID), jnp.float32),
    )(features, att, W, b)


def _make_edge_kernel(n, e, ch):
    epw = e // NW
    nch = epw // ch
    mesh = plsc.VectorSubcoreMesh(core_axis_name="c", subcore_axis_name="s")

    @functools.partial(
        pl.kernel, mesh=mesh,
        compiler_params=pltpu.CompilerParams(
            needs_layout_passes=False, use_tc_tiling_on_sc=False),
        out_type=jax.ShapeDtypeStruct((NUM_GRAPH, e), jnp.float32),
        scratch_types=[
            pltpu.VMEM((n * TW,), jnp.float32),
            pltpu.VMEM((epw,), jnp.int32),
            pltpu.VMEM((epw,), jnp.int32),
            pltpu.VMEM((2 * NUM_GRAPH * ch,), jnp.float32),
            pltpu.SemaphoreType.DMA,
            pltpu.SemaphoreType.DMA,
        ],
    )
    def edge_kernel(alr_hbm, src_hbm, dst_hbm, out_hbm, alr_v, src_v, dst_v,
                    out_v, sem_in, sem_out):
        wid = lax.axis_index("s") * NC + lax.axis_index("c")
        base0 = wid * epw
        h_tab = pltpu.async_copy(alr_hbm, alr_v, sem_in)
        h_src = pltpu.async_copy(src_hbm.at[pl.ds(base0, epw)], src_v, sem_in)
        h_dst = pltpu.async_copy(dst_hbm.at[pl.ds(base0, epw)], dst_v, sem_in)
        h_tab.wait()
        h_src.wait()
        h_dst.wait()
        out_handles = {}
        for c in range(nch):
            buf = (c % 2) * NUM_GRAPH * ch
            if c >= 2:
                for h in out_handles.pop(c - 2):
                    h.wait()

            @plsc.parallel_loop(0, ch, 16, unroll=8)
            def body(i):
                s = src_v[pl.ds(c * ch + i, 16)] * TW
                d = dst_v[pl.ds(c * ch + i, 16)] * TW + NUM_GRAPH
                for g in range(NUM_GRAPH):
                    av = plsc.load_gather(alr_v, [s + g])
                    rv = plsc.load_gather(alr_v, [d + g])
                    x = av + rv
                    out_v[pl.ds(buf + g * ch + i, 16)] = (
                        1.0 / (1.0 + jnp.exp(-x)))

            out_handles[c] = [
                pltpu.async_copy(out_v.at[pl.ds(buf + g * ch, ch)],
                                 out_hbm.at[g, pl.ds(base0 + c * ch, ch)],
                                 sem_out)
                for g in range(NUM_GRAPH)]
        for hs in out_handles.values():
            for h in hs:
                h.wait()

    return edge_kernel


def kernel(features, edge_index, att, W, b, Wl, bl, Wr, br):
    n = features.shape[0]
    e = edge_index.shape[1]
    alr = _alr_part(features, att, W, b, Wl, bl, Wr, br)
    edge_kernel = _make_edge_kernel(n, e, 2000)
    factors_t = edge_kernel(alr.reshape(-1), edge_index[0], edge_index[1])
    hidden = _hidden_part(features, att, W, b)
    return hidden, factors_t.T


# speedup vs baseline: 1.2892x; 1.2892x over previous
import functools

import jax
import jax.numpy as jnp
from jax import lax
from jax.experimental import pallas as pl
from jax.experimental.pallas import tpu as pltpu
from jax.experimental.pallas import tpu_sc as plsc

NUM_GRAPH = 4
HID = 32
SIGMA = 1.0
TW = 2 * NUM_GRAPH + 1

NC = 2
NS = 16
NW = NC * NS


def _tc_dense_kernel(f_ref, ei_ref, att_ref, w_ref, b_ref, wl_ref, bl_ref,
                     wr_ref, br_ref, hid_ref, alr_ref, src_ref, dst_ref):
    cols = []
    consts = []
    for g in range(NUM_GRAPH):
        wg = att_ref[g, :][:, None] * w_ref[g]
        cols.append(wg)
    for wv, bv in ((wl_ref, bl_ref), (wr_ref, br_ref)):
        for g in range(NUM_GRAPH):
            vg = jnp.dot(cols[g], wv[g],
                         preferred_element_type=jnp.float32)
            cols.append(vg)
            consts.append(jnp.dot(b_ref[g, :][None, :], wv[g],
                                  preferred_element_type=jnp.float32)[0, 0]
                          + bv[g][0])
    big = jnp.concatenate(cols, axis=1)
    f = f_ref[...]
    out = jnp.dot(f, big, preferred_element_type=jnp.float32)
    hid_ref[...] = out[:, :NUM_GRAPH * HID] + jnp.concatenate(
        [jnp.broadcast_to(b_ref[g, :][None, :], (f.shape[0], HID))
         for g in range(NUM_GRAPH)], axis=1)
    base = NUM_GRAPH * HID
    for g in range(2 * NUM_GRAPH):
        alr_ref[:, g:g + 1] = out[:, base + g:base + g + 1] + consts[g]
    alr_ref[:, 2 * NUM_GRAPH:] = jnp.zeros_like(
        alr_ref[:, 2 * NUM_GRAPH:])
    i = pl.program_id(0)
    eblk = ei_ref.shape[1]
    src_ref[pl.ds(i * eblk, eblk)] = ei_ref[0, :]
    dst_ref[pl.ds(i * eblk, eblk)] = ei_ref[1, :]


def _dense_part(features, edge_index, att, W, b, Wl, bl, Wr, br):
    n, d = features.shape
    e = edge_index.shape[1]
    blk = 1000
    grid = n // blk
    eblk = e // grid
    full = lambda *dims: pl.BlockSpec(dims, lambda i: (0,) * len(dims))
    return pl.pallas_call(
        _tc_dense_kernel,
        grid=(grid,),
        in_specs=[
            pl.BlockSpec((blk, d), lambda i: (i, 0)),
            pl.BlockSpec((2, eblk), lambda i: (0, i)),
            full(NUM_GRAPH, d),
            full(NUM_GRAPH, d, HID),
            full(NUM_GRAPH, HID),
            full(NUM_GRAPH, HID, 1),
            full(NUM_GRAPH, 1),
            full(NUM_GRAPH, HID, 1),
            full(NUM_GRAPH, 1),
        ],
        out_specs=[
            pl.BlockSpec((blk, NUM_GRAPH * HID), lambda i: (i, 0)),
            pl.BlockSpec((blk, TW), lambda i: (i, 0)),
            pl.BlockSpec((e,), lambda i: (0,)),
            pl.BlockSpec((e,), lambda i: (0,)),
        ],
        out_shape=[
            jax.ShapeDtypeStruct((n, NUM_GRAPH * HID), jnp.float32),
            jax.ShapeDtypeStruct((n, TW), jnp.float32),
            jax.ShapeDtypeStruct((e,), jnp.int32),
            jax.ShapeDtypeStruct((e,), jnp.int32),
        ],
    )(features, edge_index, att, W, b, Wl, bl, Wr, br)


def _make_edge_kernel(n, e, ch):
    epw = e // NW
    nch = epw // ch
    mesh = plsc.VectorSubcoreMesh(core_axis_name="c", subcore_axis_name="s")

    @functools.partial(
        pl.kernel, mesh=mesh,
        compiler_params=pltpu.CompilerParams(
            needs_layout_passes=False, use_tc_tiling_on_sc=False),
        out_type=jax.ShapeDtypeStruct((NUM_GRAPH, e), jnp.float32),
        scratch_types=[
            pltpu.VMEM((n * TW,), jnp.float32),
            pltpu.VMEM((epw,), jnp.int32),
            pltpu.VMEM((epw,), jnp.int32),
            pltpu.VMEM((2 * NUM_GRAPH * ch,), jnp.float32),
            pltpu.SemaphoreType.DMA,
            pltpu.SemaphoreType.DMA,
        ],
    )
    def edge_kernel(alr_hbm, src_hbm, dst_hbm, out_hbm, alr_v, src_v, dst_v,
                    out_v, sem_in, sem_out):
        wid = lax.axis_index("s") * NC + lax.axis_index("c")
        base0 = wid * epw
        h_tab = pltpu.async_copy(alr_hbm, alr_v, sem_in)
        h_src = pltpu.async_copy(src_hbm.at[pl.ds(base0, epw)], src_v, sem_in)
        h_dst = pltpu.async_copy(dst_hbm.at[pl.ds(base0, epw)], dst_v, sem_in)
        h_tab.wait()
        h_src.wait()
        h_dst.wait()
        out_handles = {}
        for c in range(nch):
            buf = (c % 2) * NUM_GRAPH * ch
            if c >= 2:
                for h in out_handles.pop(c - 2):
                    h.wait()

            @plsc.parallel_loop(0, ch, 16, unroll=8)
            def body(i):
                s = src_v[pl.ds(c * ch + i, 16)] * TW
                d = dst_v[pl.ds(c * ch + i, 16)] * TW + NUM_GRAPH
                for g in range(NUM_GRAPH):
                    av = plsc.load_gather(alr_v, [s + g])
                    rv = plsc.load_gather(alr_v, [d + g])
                    x = av + rv
                    out_v[pl.ds(buf + g * ch + i, 16)] = (
                        1.0 / (1.0 + jnp.exp(-x)))

            out_handles[c] = [
                pltpu.async_copy(out_v.at[pl.ds(buf + g * ch, ch)],
                                 out_hbm.at[g, pl.ds(base0 + c * ch, ch)],
                                 sem_out)
                for g in range(NUM_GRAPH)]
        for hs in out_handles.values():
            for h in hs:
                h.wait()

    return edge_kernel


def kernel(features, edge_index, att, W, b, Wl, bl, Wr, br):
    n = features.shape[0]
    e = edge_index.shape[1]
    hidden, alr, src, dst = _dense_part(
        features, edge_index, att, W, b, Wl, bl, Wr, br)
    edge_kernel = _make_edge_kernel(n, e, 2000)
    factors_t = edge_kernel(alr.reshape(-1), src, dst)
    return hidden, factors_t.T


# speedup vs baseline: 1.4619x; 1.1339x over previous
import functools

import jax
import jax.numpy as jnp
from jax import lax
from jax.experimental import pallas as pl
from jax.experimental.pallas import tpu as pltpu
from jax.experimental.pallas import tpu_sc as plsc

NUM_GRAPH = 4
HID = 32
SIGMA = 1.0
TW = 2 * NUM_GRAPH + 1
EB = 128

NC = 2
NS = 16
NW = NC * NS


def _tc_dense_kernel(f_ref, ei_ref, big_ref, bcat_ref, consts_ref,
                     hid_ref, alr_ref, src_ref, dst_ref):
    f = f_ref[...]
    out = jnp.dot(f, big_ref[...], preferred_element_type=jnp.float32)
    hid_ref[...] = out[:, :NUM_GRAPH * HID] + bcat_ref[0:1, :]
    base = NUM_GRAPH * HID
    for g in range(2 * NUM_GRAPH):
        alr_ref[:, g:g + 1] = (out[:, base + g:base + g + 1]
                               + consts_ref[0:1, g:g + 1])
    alr_ref[:, 2 * NUM_GRAPH:] = jnp.zeros_like(alr_ref[:, 2 * NUM_GRAPH:])
    i = pl.program_id(0)
    eblk = ei_ref.shape[1]
    src_ref[pl.ds(i * eblk, eblk)] = ei_ref[0, :]
    dst_ref[pl.ds(i * eblk, eblk)] = ei_ref[1, :]


def _dense_part(features, edge_index, big, bcat, consts):
    n, d = features.shape
    e = edge_index.shape[1]
    blk = 1000
    grid = n // blk
    eblk = e // grid
    wide = big.shape[1]
    full = lambda *dims: pl.BlockSpec(dims, lambda i: (0,) * len(dims))
    return pl.pallas_call(
        _tc_dense_kernel,
        grid=(grid,),
        in_specs=[
            pl.BlockSpec((blk, d), lambda i: (i, 0)),
            pl.BlockSpec((2, eblk), lambda i: (0, i)),
            full(d, wide),
            full(1, NUM_GRAPH * HID),
            full(1, 2 * NUM_GRAPH),
        ],
        out_specs=[
            pl.BlockSpec((blk, NUM_GRAPH * HID), lambda i: (i, 0)),
            pl.BlockSpec((blk, TW), lambda i: (i, 0)),
            pl.BlockSpec((e,), lambda i: (0,)),
            pl.BlockSpec((e,), lambda i: (0,)),
        ],
        out_shape=[
            jax.ShapeDtypeStruct((n, NUM_GRAPH * HID), jnp.float32),
            jax.ShapeDtypeStruct((n, TW), jnp.float32),
            jax.ShapeDtypeStruct((e,), jnp.int32),
            jax.ShapeDtypeStruct((e,), jnp.int32),
        ],
    )(features, edge_index, big, bcat, consts)


def _make_edge_kernel(n, e):
    neb = e // EB
    epb = neb // NW
    extra = neb - NW * epb
    cpw = 6
    bpc = epb // cpw
    ch_e = bpc * EB
    ch_w = bpc * EB * NUM_GRAPH
    mesh = plsc.VectorSubcoreMesh(core_axis_name="c", subcore_axis_name="s")

    @functools.partial(
        pl.kernel, mesh=mesh,
        compiler_params=pltpu.CompilerParams(
            needs_layout_passes=False, use_tc_tiling_on_sc=False),
        out_type=jax.ShapeDtypeStruct((e * NUM_GRAPH,), jnp.float32),
        scratch_types=[
            pltpu.VMEM((n * TW,), jnp.float32),
            pltpu.VMEM((epb * EB,), jnp.int32),
            pltpu.VMEM((epb * EB,), jnp.int32),
            pltpu.VMEM((2 * ch_w,), jnp.float32),
            pltpu.VMEM((EB,), jnp.int32),
            pltpu.VMEM((EB,), jnp.int32),
            pltpu.VMEM((EB * NUM_GRAPH,), jnp.float32),
            pltpu.SemaphoreType.DMA,
            pltpu.SemaphoreType.DMA,
        ],
    )
    def edge_kernel(alr_hbm, src_hbm, dst_hbm, out_hbm, alr_v, src_v, dst_v,
                    out_v, xsrc_v, xdst_v, xout_v, sem_in, sem_out):
        wid = lax.axis_index("s") * NC + lax.axis_index("c")
        e0 = wid * epb * EB
        h_tab = pltpu.async_copy(alr_hbm, alr_v, sem_in)
        h_src = pltpu.async_copy(src_hbm.at[pl.ds(e0, epb * EB)], src_v,
                                 sem_in)
        h_dst = pltpu.async_copy(dst_hbm.at[pl.ds(e0, epb * EB)], dst_v,
                                 sem_in)
        h_tab.wait()
        h_src.wait()
        h_dst.wait()
        out_handles = {}
        for c in range(cpw):
            buf = (c % 2) * ch_w
            if c >= 2:
                out_handles.pop(c - 2).wait()

            @plsc.parallel_loop(0, ch_e, 16, unroll=8)
            def body(i):
                s = src_v[pl.ds(c * ch_e + i, 16)] * TW
                d = dst_v[pl.ds(c * ch_e + i, 16)] * TW + NUM_GRAPH
                ob = buf + (i // EB) * (EB * NUM_GRAPH) + (i % EB)
                for g in range(NUM_GRAPH):
                    av = plsc.load_gather(alr_v, [s + g])
                    rv = plsc.load_gather(alr_v, [d + g])
                    x = av + rv
                    out_v[pl.ds(ob + g * EB, 16)] = 1.0 / (1.0 + jnp.exp(-x))

            out_handles[c] = pltpu.async_copy(
                out_v.at[pl.ds(buf, ch_w)],
                out_hbm.at[pl.ds((e0 + c * ch_e) * NUM_GRAPH, ch_w)],
                sem_out)

        @pl.when(wid < extra)
        def _():
            xe0 = (NW * epb + wid) * EB
            hs = pltpu.async_copy(src_hbm.at[pl.ds(xe0, EB)], xsrc_v, sem_in)
            hd = pltpu.async_copy(dst_hbm.at[pl.ds(xe0, EB)], xdst_v, sem_in)
            hs.wait()
            hd.wait()

            @plsc.parallel_loop(0, EB, 16, unroll=8)
            def xbody(i):
                s = xsrc_v[pl.ds(i, 16)] * TW
                d = xdst_v[pl.ds(i, 16)] * TW + NUM_GRAPH
                for g in range(NUM_GRAPH):
                    av = plsc.load_gather(alr_v, [s + g])
                    rv = plsc.load_gather(alr_v, [d + g])
                    x = av + rv
                    xout_v[pl.ds(g * EB + i, 16)] = 1.0 / (1.0 + jnp.exp(-x))

            pltpu.sync_copy(xout_v,
                            out_hbm.at[pl.ds(xe0 * NUM_GRAPH,
                                             EB * NUM_GRAPH)])

        for h in out_handles.values():
            h.wait()

    return edge_kernel


def kernel(features, edge_index, att, W, b, Wl, bl, Wr, br):
    n = features.shape[0]
    e = edge_index.shape[1]
    wg = att[:, :, None] * W
    w_eff = jnp.transpose(wg, (1, 0, 2)).reshape(features.shape[1],
                                                 NUM_GRAPH * HID)
    vl = jnp.einsum("gdh,gho->dg", wg, Wl)
    vr = jnp.einsum("gdh,gho->dg", wg, Wr)
    big = jnp.concatenate([w_eff, vl, vr], axis=1)
    bcat = b.reshape(1, NUM_GRAPH * HID)
    cl = jnp.einsum("gh,gho->g", b, Wl) + bl[:, 0]
    cr = jnp.einsum("gh,gho->g", b, Wr) + br[:, 0]
    consts = jnp.concatenate([cl, cr]).reshape(1, 2 * NUM_GRAPH)

    hidden, alr, src, dst = _dense_part(features, edge_index, big, bcat,
                                        consts)
    edge_kernel = _make_edge_kernel(n, e)
    out_flat = edge_kernel(alr.reshape(-1), src, dst)
    factors = (out_flat.reshape(e // EB, NUM_GRAPH, EB)
               .transpose(0, 2, 1).reshape(e, NUM_GRAPH))
    return hidden, factors


# speedup vs baseline: 1.4807x; 1.0128x over previous
import functools

import jax
import jax.numpy as jnp
from jax import lax
from jax.experimental import pallas as pl
from jax.experimental.pallas import tpu as pltpu
from jax.experimental.pallas import tpu_sc as plsc

NUM_GRAPH = 4
HID = 32
SIGMA = 1.0
TW = 2 * NUM_GRAPH + 1
EB = 128

NC = 2
NS = 16
NW = NC * NS


def _tc_alr_kernel(f_ref, ei_ref, vlr_ref, consts_ref,
                   alr_ref, src_ref, dst_ref):
    f = f_ref[...]
    out = jnp.dot(f, vlr_ref[...], preferred_element_type=jnp.float32)
    for g in range(2 * NUM_GRAPH):
        alr_ref[:, g:g + 1] = (out[:, g:g + 1] + consts_ref[0:1, g:g + 1])
    alr_ref[:, 2 * NUM_GRAPH:] = jnp.zeros_like(alr_ref[:, 2 * NUM_GRAPH:])
    i = pl.program_id(0)
    eblk = ei_ref.shape[1]
    src_ref[pl.ds(i * eblk, eblk)] = ei_ref[0, :]
    dst_ref[pl.ds(i * eblk, eblk)] = ei_ref[1, :]


def _alr_part(features, edge_index, vlr, consts):
    n, d = features.shape
    e = edge_index.shape[1]
    blk = 1000
    grid = n // blk
    eblk = e // grid
    full = lambda *dims: pl.BlockSpec(dims, lambda i: (0,) * len(dims))
    return pl.pallas_call(
        _tc_alr_kernel,
        grid=(grid,),
        in_specs=[
            pl.BlockSpec((blk, d), lambda i: (i, 0)),
            pl.BlockSpec((2, eblk), lambda i: (0, i)),
            full(d, 2 * NUM_GRAPH),
            full(1, 2 * NUM_GRAPH),
        ],
        out_specs=[
            pl.BlockSpec((blk, TW), lambda i: (i, 0)),
            pl.BlockSpec((e,), lambda i: (0,)),
            pl.BlockSpec((e,), lambda i: (0,)),
        ],
        out_shape=[
            jax.ShapeDtypeStruct((n, TW), jnp.float32),
            jax.ShapeDtypeStruct((e,), jnp.int32),
            jax.ShapeDtypeStruct((e,), jnp.int32),
        ],
    )(features, edge_index, vlr, consts)


def _tc_hidden_kernel(f_ref, weff_ref, bcat_ref, hid_ref):
    f = f_ref[...]
    out = jnp.dot(f, weff_ref[...], preferred_element_type=jnp.float32)
    hid_ref[...] = out + bcat_ref[0:1, :]


def _hidden_part(features, weff, bcat):
    n, d = features.shape
    blk = 1000
    grid = n // blk
    full = lambda *dims: pl.BlockSpec(dims, lambda i: (0,) * len(dims))
    return pl.pallas_call(
        _tc_hidden_kernel,
        grid=(grid,),
        in_specs=[
            pl.BlockSpec((blk, d), lambda i: (i, 0)),
            full(d, NUM_GRAPH * HID),
            full(1, NUM_GRAPH * HID),
        ],
        out_specs=pl.BlockSpec((blk, NUM_GRAPH * HID), lambda i: (i, 0)),
        out_shape=jax.ShapeDtypeStruct((n, NUM_GRAPH * HID), jnp.float32),
    )(features, weff, bcat)


def _make_edge_kernel(n, e):
    neb = e // EB
    epb = neb // NW
    extra = neb - NW * epb
    cpw = 6
    bpc = epb // cpw
    ch_e = bpc * EB
    ch_w = bpc * EB * NUM_GRAPH
    mesh = plsc.VectorSubcoreMesh(core_axis_name="c", subcore_axis_name="s")

    @functools.partial(
        pl.kernel, mesh=mesh,
        compiler_params=pltpu.CompilerParams(
            needs_layout_passes=False, use_tc_tiling_on_sc=False),
        out_type=jax.ShapeDtypeStruct((e * NUM_GRAPH,), jnp.float32),
        scratch_types=[
            pltpu.VMEM((n * TW,), jnp.float32),
            pltpu.VMEM((epb * EB,), jnp.int32),
            pltpu.VMEM((epb * EB,), jnp.int32),
            pltpu.VMEM((2 * ch_w,), jnp.float32),
            pltpu.VMEM((EB,), jnp.int32),
            pltpu.VMEM((EB,), jnp.int32),
            pltpu.VMEM((EB * NUM_GRAPH,), jnp.float32),
            pltpu.SemaphoreType.DMA,
            pltpu.SemaphoreType.DMA,
        ],
    )
    def edge_kernel(alr_hbm, src_hbm, dst_hbm, out_hbm, alr_v, src_v, dst_v,
                    out_v, xsrc_v, xdst_v, xout_v, sem_in, sem_out):
        wid = lax.axis_index("s") * NC + lax.axis_index("c")
        e0 = wid * epb * EB
        h_tab = pltpu.async_copy(alr_hbm, alr_v, sem_in)
        h_src = pltpu.async_copy(src_hbm.at[pl.ds(e0, epb * EB)], src_v,
                                 sem_in)
        h_dst = pltpu.async_copy(dst_hbm.at[pl.ds(e0, epb * EB)], dst_v,
                                 sem_in)
        h_tab.wait()
        h_src.wait()
        h_dst.wait()
        out_handles = {}
        for c in range(cpw):
            buf = (c % 2) * ch_w
            if c >= 2:
                out_handles.pop(c - 2).wait()

            @plsc.parallel_loop(0, ch_e, 16, unroll=8)
            def body(i):
                s = src_v[pl.ds(c * ch_e + i, 16)] * TW
                d = dst_v[pl.ds(c * ch_e + i, 16)] * TW + NUM_GRAPH
                ob = buf + (i // EB) * (EB * NUM_GRAPH) + (i % EB)
                for g in range(NUM_GRAPH):
                    av = plsc.load_gather(alr_v, [s + g])
                    rv = plsc.load_gather(alr_v, [d + g])
                    x = av + rv
                    out_v[pl.ds(ob + g * EB, 16)] = 1.0 / (1.0 + jnp.exp(-x))

            out_handles[c] = pltpu.async_copy(
                out_v.at[pl.ds(buf, ch_w)],
                out_hbm.at[pl.ds((e0 + c * ch_e) * NUM_GRAPH, ch_w)],
                sem_out)

        @pl.when(wid < extra)
        def _():
            xe0 = (NW * epb + wid) * EB
            hs = pltpu.async_copy(src_hbm.at[pl.ds(xe0, EB)], xsrc_v, sem_in)
            hd = pltpu.async_copy(dst_hbm.at[pl.ds(xe0, EB)], xdst_v, sem_in)
            hs.wait()
            hd.wait()

            @plsc.parallel_loop(0, EB, 16, unroll=8)
            def xbody(i):
                s = xsrc_v[pl.ds(i, 16)] * TW
                d = xdst_v[pl.ds(i, 16)] * TW + NUM_GRAPH
                for g in range(NUM_GRAPH):
                    av = plsc.load_gather(alr_v, [s + g])
                    rv = plsc.load_gather(alr_v, [d + g])
                    x = av + rv
                    xout_v[pl.ds(g * EB + i, 16)] = 1.0 / (1.0 + jnp.exp(-x))

            pltpu.sync_copy(xout_v,
                            out_hbm.at[pl.ds(xe0 * NUM_GRAPH,
                                             EB * NUM_GRAPH)])

        for h in out_handles.values():
            h.wait()

    return edge_kernel


def kernel(features, edge_index, att, W, b, Wl, bl, Wr, br):
    n = features.shape[0]
    e = edge_index.shape[1]
    wg = att[:, :, None] * W
    w_eff = jnp.transpose(wg, (1, 0, 2)).reshape(features.shape[1],
                                                 NUM_GRAPH * HID)
    vl = jnp.einsum("gdh,gho->dg", wg, Wl)
    vr = jnp.einsum("gdh,gho->dg", wg, Wr)
    vlr = jnp.concatenate([vl, vr], axis=1)
    bcat = b.reshape(1, NUM_GRAPH * HID)
    cl = jnp.einsum("gh,gho->g", b, Wl) + bl[:, 0]
    cr = jnp.einsum("gh,gho->g", b, Wr) + br[:, 0]
    consts = jnp.concatenate([cl, cr]).reshape(1, 2 * NUM_GRAPH)

    alr, src, dst = _alr_part(features, edge_index, vlr, consts)
    edge_kernel = _make_edge_kernel(n, e)
    out_flat = edge_kernel(alr.reshape(-1), src, dst)
    hidden = _hidden_part(features, w_eff, bcat)
    factors = (out_flat.reshape(e // EB, NUM_GRAPH, EB)
               .transpose(0, 2, 1).reshape(e, NUM_GRAPH))
    return hidden, factors
